# baseline (speedup 1.0000x reference)
import functools

import jax
import jax.numpy as jnp
from jax import lax
from jax.experimental import pallas as pl
from jax.experimental.pallas import tpu as pltpu
from jax.experimental.pallas import tpu_sc as plsc

EMB = 32
OUT_DIM = 128

NC = 2
NS = 16
NW = NC * NS

CHUNK = 128
QB = 16384


def _repack_body(tT_ref, o_ref):
  x = tT_ref[...]
  xx = jnp.concatenate([x[:, u * QB:(u + 1) * QB] for u in range(4)], axis=0)
  r = lax.broadcasted_iota(jnp.int32, (4 * EMB, 4 * EMB), 0)
  c = lax.broadcasted_iota(jnp.int32, (4 * EMB, 4 * EMB), 1)
  eye = jnp.where(r == c, 1.0, 0.0)
  o_ref[...] = lax.dot_general(
      xx, eye, (((0,), (0,)), ((), ())), preferred_element_type=jnp.float32)


def _tc_repack(tableT):
  vocab = tableT.shape[1]
  grid = pl.cdiv(vocab, 4 * QB)
  return pl.pallas_call(
      _repack_body,
      grid=(grid,),
      in_specs=[pl.BlockSpec((EMB, 4 * QB), lambda i: (0, i))],
      out_specs=pl.BlockSpec((QB, 4 * EMB), lambda i: (i, 0)),
      out_shape=jax.ShapeDtypeStruct((grid * QB, 4 * EMB), jnp.float32),
  )(tableT)


NBUF = 5


def _gather_body(idx_hbm, table_hbm, out_hbm, idx_v, m_v, rows_bufs, sems,
                 rows_per_w, n_chunks, n_tok):
  wid = lax.axis_index("s") * NC + lax.axis_index("c")
  rq = rows_per_w // 4
  for u in range(4):
    pltpu.sync_copy(
        idx_hbm.at[pl.ds(u * (n_tok // 4) + wid * rq, rq)],
        idx_v.at[pl.ds(u * rq, rq)])

  @pl.loop(0, rows_per_w // 16)
  def _remap(j):
    pl0 = j * 16
    lane = lax.iota(jnp.int32, 16) + pl0
    g = (lane & 3) * rq + (lane >> 2)
    v = plsc.load_gather(idx_v, [g])
    i = v >> 16
    u = (v >> 14) & 3
    q = v & 16383
    m_v[pl.ds(pl0, 16)] = (((i << 14) | q) << 2) | u

  base = wid * rows_per_w

  @pl.loop(0, n_chunks, step=NBUF)
  def _chunks(c):
    handles = []
    for k in range(NBUF):
      handles.append(
          pltpu.async_copy(
              table_hbm.at[m_v.at[pl.ds((c + k) * CHUNK, CHUNK)]],
              rows_bufs[k], sems[k]))
    for k in range(NBUF):
      handles[k].wait()
      pltpu.sync_copy(rows_bufs[k],
                      out_hbm.at[pl.ds(base + (c + k) * CHUNK, CHUNK)])


def _sc_gather(idx_flat, table32):
  n = idx_flat.shape[0]
  rows_per_w = n // NW
  n_chunks = rows_per_w // CHUNK
  mesh = plsc.VectorSubcoreMesh(
      core_axis_name="c", subcore_axis_name="s", num_cores=NC,
      num_subcores=NS)
  body = functools.partial(
      _gather_body, rows_per_w=rows_per_w, n_chunks=n_chunks, n_tok=n)
  return pl.kernel(
      body,
      out_type=jax.ShapeDtypeStruct((n, EMB), jnp.float32),
      mesh=mesh,
      scratch_types=[
          pltpu.VMEM((rows_per_w,), jnp.int32),
          pltpu.VMEM((rows_per_w,), jnp.int32),
          [pltpu.VMEM((CHUNK, EMB), jnp.float32) for _ in range(NBUF)],
          [pltpu.SemaphoreType.DMA for _ in range(NBUF)],
      ],
      compiler_params=pltpu.CompilerParams(
          use_tc_tiling_on_sc=False, needs_layout_passes=False),
  )(idx_flat, table32)


def _decode_body(x_ref, w_ref, b_ref, o_ref):
  x = x_ref[...]
  for u in range(4):
    o_ref[u] = jnp.dot(
        x[:, u * EMB:(u + 1) * EMB], w_ref[...],
        preferred_element_type=jnp.float32) + b_ref[...]


def _tc_decode(embeds4, w, b):
  n4 = embeds4.shape[0]
  rb = 10240
  return pl.pallas_call(
      _decode_body,
      grid=(n4 // rb,),
      in_specs=[
          pl.BlockSpec((rb, 4 * EMB), lambda i: (i, 0)),
          pl.BlockSpec((EMB, OUT_DIM), lambda i: (0, 0)),
          pl.BlockSpec((1, OUT_DIM), lambda i: (0, 0)),
      ],
      out_specs=pl.BlockSpec((4, rb, OUT_DIM), lambda i: (0, i, 0)),
      out_shape=jax.ShapeDtypeStruct((4, n4, OUT_DIM), jnp.float32),
  )(embeds4, w, b)


@jax.jit
def kernel(input_ids, emb_weights, W_dec, b_dec):
  bsz, seq = input_ids.shape
  n = bsz * seq
  idx_flat = input_ids.reshape(-1)
  packed = _tc_repack(emb_weights.T)
  table32 = packed.reshape(-1, EMB)
  embeds = _sc_gather(idx_flat, table32)
  out = _tc_decode(embeds.reshape(n // 4, 4 * EMB), W_dec,
                   b_dec.reshape(1, OUT_DIM))
  return out.reshape(bsz, seq, OUT_DIM)


# speedup vs baseline: 1.0235x; 1.0235x over previous
import functools

import jax
import jax.numpy as jnp
from jax import lax
from jax.experimental import pallas as pl
from jax.experimental.pallas import tpu as pltpu
from jax.experimental.pallas import tpu_sc as plsc

EMB = 32
OUT_DIM = 128

NC = 2
NS = 16
NW = NC * NS

CHUNK = 128
QB = 16384


def _repack_body(tT_ref, o_ref):
  x = tT_ref[...]
  xx = jnp.concatenate([x[:, u * QB:(u + 1) * QB] for u in range(4)], axis=0)
  r = lax.broadcasted_iota(jnp.int32, (4 * EMB, 4 * EMB), 0)
  c = lax.broadcasted_iota(jnp.int32, (4 * EMB, 4 * EMB), 1)
  eye = jnp.where(r == c, 1.0, 0.0)
  o_ref[...] = lax.dot_general(
      xx, eye, (((0,), (0,)), ((), ())), preferred_element_type=jnp.float32)


def _tc_repack(tableT):
  vocab = tableT.shape[1]
  grid = pl.cdiv(vocab, 4 * QB)
  return pl.pallas_call(
      _repack_body,
      grid=(grid,),
      in_specs=[pl.BlockSpec((EMB, 4 * QB), lambda i: (0, i))],
      out_specs=pl.BlockSpec((QB, 4 * EMB), lambda i: (i, 0)),
      out_shape=jax.ShapeDtypeStruct((grid * QB, 4 * EMB), jnp.float32),
  )(tableT)


NBUF = 10


def _gather_body(idx_hbm, table_hbm, out_hbm, idx_v, m_v, rows_bufs, sems,
                 rows_per_w, n_chunks, n_tok):
  wid = lax.axis_index("s") * NC + lax.axis_index("c")
  rq = rows_per_w // 4
  for u in range(4):
    pltpu.sync_copy(
        idx_hbm.at[pl.ds(u * (n_tok // 4) + wid * rq, rq)],
        idx_v.at[pl.ds(u * rq, rq)])

  @pl.loop(0, rows_per_w // 16)
  def _remap(j):
    pl0 = j * 16
    lane = lax.iota(jnp.int32, 16) + pl0
    g = (lane & 3) * rq + (lane >> 2)
    v = plsc.load_gather(idx_v, [g])
    i = v >> 16
    u = (v >> 14) & 3
    q = v & 16383
    m_v[pl.ds(pl0, 16)] = (((i << 14) | q) << 2) | u

  base = wid * rows_per_w

  @pl.loop(0, n_chunks, step=NBUF)
  def _chunks(c):
    handles = []
    for k in range(NBUF):
      handles.append(
          pltpu.async_copy(
              table_hbm.at[m_v.at[pl.ds((c + k) * CHUNK, CHUNK)]],
              rows_bufs[k], sems[k]))
    for k in range(NBUF):
      handles[k].wait()
      pltpu.sync_copy(rows_bufs[k],
                      out_hbm.at[pl.ds(base + (c + k) * CHUNK, CHUNK)])


def _sc_gather(idx_flat, table32):
  n = idx_flat.shape[0]
  rows_per_w = n // NW
  n_chunks = rows_per_w // CHUNK
  mesh = plsc.VectorSubcoreMesh(
      core_axis_name="c", subcore_axis_name="s", num_cores=NC,
      num_subcores=NS)
  body = functools.partial(
      _gather_body, rows_per_w=rows_per_w, n_chunks=n_chunks, n_tok=n)
  return pl.kernel(
      body,
      out_type=jax.ShapeDtypeStruct((n, EMB), jnp.float32),
      mesh=mesh,
      scratch_types=[
          pltpu.VMEM((rows_per_w,), jnp.int32),
          pltpu.VMEM((rows_per_w,), jnp.int32),
          [pltpu.VMEM((CHUNK, EMB), jnp.float32) for _ in range(NBUF)],
          [pltpu.SemaphoreType.DMA for _ in range(NBUF)],
      ],
      compiler_params=pltpu.CompilerParams(
          use_tc_tiling_on_sc=False, needs_layout_passes=False),
  )(idx_flat, table32)


def _decode_body(x_ref, w_ref, b_ref, o_ref):
  x = x_ref[...]
  for u in range(4):
    o_ref[u] = jnp.dot(
        x[:, u * EMB:(u + 1) * EMB], w_ref[...],
        preferred_element_type=jnp.float32) + b_ref[...]


def _tc_decode(embeds4, w, b):
  n4 = embeds4.shape[0]
  rb = 10240
  return pl.pallas_call(
      _decode_body,
      grid=(n4 // rb,),
      in_specs=[
          pl.BlockSpec((rb, 4 * EMB), lambda i: (i, 0)),
          pl.BlockSpec((EMB, OUT_DIM), lambda i: (0, 0)),
          pl.BlockSpec((1, OUT_DIM), lambda i: (0, 0)),
      ],
      out_specs=pl.BlockSpec((4, rb, OUT_DIM), lambda i: (0, i, 0)),
      out_shape=jax.ShapeDtypeStruct((4, n4, OUT_DIM), jnp.float32),
  )(embeds4, w, b)


@jax.jit
def kernel(input_ids, emb_weights, W_dec, b_dec):
  bsz, seq = input_ids.shape
  n = bsz * seq
  idx_flat = input_ids.reshape(-1)
  packed = _tc_repack(emb_weights.T)
  table32 = packed.reshape(-1, EMB)
  embeds = _sc_gather(idx_flat, table32)
  out = _tc_decode(embeds.reshape(n // 4, 4 * EMB), W_dec,
                   b_dec.reshape(1, OUT_DIM))
  return out.reshape(bsz, seq, OUT_DIM)
